# baseline (speedup 1.0000x reference)
import functools

import jax
import jax.numpy as jnp
from jax import lax
from jax.experimental import pallas as pl
from jax.experimental.pallas import tpu as pltpu
from jax.experimental.pallas import tpu_sc as plsc


def _lse_body(table_ref, out_ref):
    x = table_ref[...]
    m = jnp.max(x, axis=1, keepdims=True)
    s = jnp.sum(jnp.exp(x - m), axis=1, keepdims=True)
    out_ref[...] = m + jnp.log(s)


def _compute_lse(table):
    v = table.shape[0]
    return pl.pallas_call(
        _lse_body,
        out_shape=jax.ShapeDtypeStruct((v, 1), jnp.float32),
    )(table)


@functools.lru_cache(maxsize=None)
def _make_sc_gather(V, D, B):
    info = plsc.get_sparse_core_info()
    NC, NS, L = info.num_cores, info.num_subcores, info.num_lanes
    NW = NC * NS
    assert B % NW == 0
    bpw = B // NW
    TB = 128
    assert bpw % TB == 0
    NBLK = bpw // TB
    SW = 256
    NS_ = (D + SW - 1) // SW
    DP = NS_ * SW
    stripe_rows = [min(SW, D - s * SW) for s in range(NS_)]
    mesh = plsc.VectorSubcoreMesh(core_axis_name="c", subcore_axis_name="s")

    @functools.partial(
        pl.kernel,
        mesh=mesh,
        compiler_params=pltpu.CompilerParams(needs_layout_passes=False),
        out_type=[
            jax.ShapeDtypeStruct((D, B), jnp.float32),
            jax.ShapeDtypeStruct((NW * L,), jnp.float32),
        ],
        scratch_types=[
            pltpu.VMEM((TB,), jnp.int32),
            pltpu.VMEM((TB,), jnp.int32),
            pltpu.VMEM((TB,), jnp.int32),
            pltpu.VMEM((2, TB), jnp.int32),
            pltpu.VMEM((2 * TB, SW), jnp.float32),
            pltpu.VMEM((SW, TB), jnp.float32),
            pltpu.VMEM((TB,), jnp.float32),
            pltpu.VMEM((TB,), jnp.float32),
            pltpu.VMEM((L,), jnp.float32),
            pltpu.SemaphoreType.DMA,
            pltpu.SemaphoreType.DMA,
            pltpu.SemaphoreType.DMA,
        ],
    )
    def sc_kernel(tab4_hbm, tflat_hbm, xb_hbm, yb_hbm, lse_hbm,
                  outt_hbm, part_hbm,
                  xidx_v, yidx_v, fidx_v, eidx_v, rows_v, rt_v,
                  lsev_v, tv_v, acc_v, sem0, sem1, lsem):
        wid = lax.axis_index("s") * NC + lax.axis_index("c")
        acc_v[...] = jnp.zeros((L,), jnp.float32)
        sems = (sem0, sem1)

        def fire_stripe(s):
            slot = s % 2
            for kk in range(TB // L):
                sl = pl.ds(kk * L, L)
                eidx_v[slot, sl] = xidx_v[sl] * NS_ + s
            pltpu.async_copy(
                tab4_hbm.at[eidx_v.at[slot]],
                rows_v.at[pl.ds(slot * TB, TB)], sems[slot])

        def drain_stripe(s):
            slot = s % 2
            pltpu.make_async_copy(
                tab4_hbm.at[eidx_v.at[slot]],
                rows_v.at[pl.ds(slot * TB, TB)], sems[slot]).wait()

        def body(blk, carry):
            tbase = wid * bpw + blk * TB
            pltpu.sync_copy(xb_hbm.at[pl.ds(tbase, TB)], xidx_v)
            pltpu.sync_copy(yb_hbm.at[pl.ds(tbase, TB)], yidx_v)
            for kk in range(TB // L):
                sl = pl.ds(kk * L, L)
                fidx_v[sl] = xidx_v[sl] * D + yidx_v[sl]
            cp_lse = pltpu.async_copy(lse_hbm.at[xidx_v], lsev_v, lsem)
            cp_tv = pltpu.async_copy(tflat_hbm.at[fidx_v], tv_v, lsem)
            fire_stripe(0)
            for s in range(NS_):
                if s + 1 < NS_:
                    fire_stripe(s + 1)
                drain_stripe(s)
                slot = s % 2
                rbase = slot * TB

                def tr_body(c, carry2):
                    for tg in range(TB // L):
                        ridx = rbase + tg * L + lax.iota(jnp.int32, L)
                        cidx = jnp.broadcast_to(c, (L,)).astype(jnp.int32)
                        vals = plsc.load_gather(rows_v, [ridx, cidx])
                        rt_v[c, pl.ds(tg * L, L)] = vals
                    return carry2

                lax.fori_loop(0, stripe_rows[s], tr_body, 0)
                pltpu.sync_copy(
                    rt_v.at[pl.ds(0, stripe_rows[s])],
                    outt_hbm.at[pl.ds(s * SW, stripe_rows[s]),
                                pl.ds(tbase, TB)])
            cp_lse.wait()
            cp_tv.wait()
            for kk in range(TB // L):
                sl = pl.ds(kk * L, L)
                acc_v[...] = acc_v[...] + (lsev_v[sl] - tv_v[sl])
            return carry

        lax.fori_loop(0, NBLK, body, 0)
        pltpu.sync_copy(acc_v, part_hbm.at[pl.ds(wid * L, L)])

    return sc_kernel


def kernel(xb, yb, table):
    Bb, Tt = xb.shape
    V, D = table.shape
    N = Bb * Tt
    SW = 256
    NS_ = (D + SW - 1) // SW
    xbf = xb.reshape(N).astype(jnp.int32)
    ybf = yb.reshape(N).astype(jnp.int32)
    lse = _compute_lse(table).reshape(V)
    tab4 = jnp.pad(table, ((0, 0), (0, NS_ * SW - D))).reshape(V * NS_, SW)
    logitsT, parts = _make_sc_gather(V, D, N)(
        tab4, table.reshape(V * D), xbf, ybf, lse)
    loss = jnp.sum(parts) / N
    return (logitsT.T, loss)


# speedup vs baseline: 1.3473x; 1.3473x over previous
import functools

import jax
import jax.numpy as jnp
from jax import lax
from jax.experimental import pallas as pl
from jax.experimental.pallas import tpu as pltpu
from jax.experimental.pallas import tpu_sc as plsc


def _lse_body(table_ref, out_ref):
    x = table_ref[...]
    m = jnp.max(x, axis=1, keepdims=True)
    s = jnp.sum(jnp.exp(x - m), axis=1, keepdims=True)
    out_ref[...] = m + jnp.log(s)


def _compute_lse(table):
    v = table.shape[0]
    return pl.pallas_call(
        _lse_body,
        out_shape=jax.ShapeDtypeStruct((v, 1), jnp.float32),
    )(table)


@functools.lru_cache(maxsize=None)
def _make_sc_gather(V, D, B):
    info = plsc.get_sparse_core_info()
    NC, NS, L = info.num_cores, info.num_subcores, info.num_lanes
    NW = NC * NS
    assert B % NW == 0
    bpw = B // NW
    TB = 128
    assert bpw % TB == 0
    NBLK = bpw // TB
    SW = 128
    NST = (D + SW - 1) // SW
    stripe_rows = [min(SW, D - s * SW) for s in range(NST)]
    mesh = plsc.VectorSubcoreMesh(core_axis_name="c", subcore_axis_name="s")

    @functools.partial(
        pl.kernel,
        mesh=mesh,
        compiler_params=pltpu.CompilerParams(needs_layout_passes=False),
        out_type=[
            jax.ShapeDtypeStruct((D, B), jnp.float32),
            jax.ShapeDtypeStruct((NW * L,), jnp.float32),
        ],
        scratch_types=[
            pltpu.VMEM((TB,), jnp.int32),
            pltpu.VMEM((TB,), jnp.int32),
            pltpu.VMEM((TB,), jnp.int32),
            pltpu.VMEM((2, TB), jnp.int32),
            pltpu.VMEM((2 * TB, SW), jnp.float32),
            pltpu.VMEM((2 * SW, TB), jnp.float32),
            pltpu.VMEM((TB,), jnp.float32),
            pltpu.VMEM((TB,), jnp.float32),
            pltpu.VMEM((L,), jnp.float32),
            pltpu.SemaphoreType.DMA,
            pltpu.SemaphoreType.DMA,
            pltpu.SemaphoreType.DMA,
            pltpu.SemaphoreType.DMA,
            pltpu.SemaphoreType.DMA,
        ],
    )
    def sc_kernel(tab8_hbm, tflat_hbm, xb_hbm, yb_hbm, lse_hbm,
                  outt_hbm, part_hbm,
                  xidx_v, yidx_v, fidx_v, eidx_v, rows_v, rt_v,
                  lsev_v, tv_v, acc_v, gsem0, gsem1, wsem0, wsem1, lsem):
        wid = lax.axis_index("s") * NC + lax.axis_index("c")
        acc_v[...] = jnp.zeros((L,), jnp.float32)
        gsems = (gsem0, gsem1)
        wsems = (wsem0, wsem1)

        def body(blk, carry):
            tbase = wid * bpw + blk * TB
            pltpu.sync_copy(xb_hbm.at[pl.ds(tbase, TB)], xidx_v)
            pltpu.sync_copy(yb_hbm.at[pl.ds(tbase, TB)], yidx_v)
            for kk in range(TB // L):
                sl = pl.ds(kk * L, L)
                fidx_v[sl] = xidx_v[sl] * D + yidx_v[sl]
            cp_lse = pltpu.async_copy(lse_hbm.at[xidx_v], lsev_v, lsem)
            cp_tv = pltpu.async_copy(tflat_hbm.at[fidx_v], tv_v, lsem)

            def fire_g(s):
                slot = s % 2
                for kk in range(TB // L):
                    sl = pl.ds(kk * L, L)
                    eidx_v[slot, sl] = xidx_v[sl] * NST + s
                pltpu.async_copy(
                    tab8_hbm.at[eidx_v.at[slot]],
                    rows_v.at[pl.ds(slot * TB, TB)], gsems[slot])

            def drain_g(s):
                slot = s % 2
                pltpu.make_async_copy(
                    tab8_hbm.at[eidx_v.at[slot]],
                    rows_v.at[pl.ds(slot * TB, TB)], gsems[slot]).wait()

            def fire_w(s):
                slot = s % 2
                pltpu.async_copy(
                    rt_v.at[pl.ds(slot * SW, stripe_rows[s])],
                    outt_hbm.at[pl.ds(s * SW, stripe_rows[s]),
                                pl.ds(tbase, TB)], wsems[slot])

            def drain_w(s):
                slot = s % 2
                pltpu.make_async_copy(
                    rt_v.at[pl.ds(slot * SW, stripe_rows[s])],
                    outt_hbm.at[pl.ds(s * SW, stripe_rows[s]),
                                pl.ds(tbase, TB)], wsems[slot]).wait()

            fire_g(0)
            for s in range(NST):
                if s + 1 < NST:
                    fire_g(s + 1)
                drain_g(s)
                if s >= 2:
                    drain_w(s - 2)
                slot = s % 2
                rbase = slot * TB
                cbase = slot * SW

                @plsc.parallel_loop(0, TB)
                def tr_body(t):
                    vals = [
                        rows_v[rbase + t, pl.ds(cg * L, L)]
                        for cg in range(SW // L)
                    ]
                    tsp = jnp.broadcast_to(t, (L,)).astype(jnp.int32)
                    for cg in range(SW // L):
                        cvec = cbase + cg * L + lax.iota(jnp.int32, L)
                        plsc.store_scatter(rt_v, [cvec, tsp], vals[cg])

                fire_w(s)
            drain_w(NST - 2)
            drain_w(NST - 1)

            cp_lse.wait()
            cp_tv.wait()
            for kk in range(TB // L):
                sl = pl.ds(kk * L, L)
                acc_v[...] = acc_v[...] + (lsev_v[sl] - tv_v[sl])
            return carry

        lax.fori_loop(0, NBLK, body, 0)
        pltpu.sync_copy(acc_v, part_hbm.at[pl.ds(wid * L, L)])

    return sc_kernel


def kernel(xb, yb, table):
    Bb, Tt = xb.shape
    V, D = table.shape
    N = Bb * Tt
    SW = 128
    NST = (D + SW - 1) // SW
    xbf = xb.reshape(N).astype(jnp.int32)
    ybf = yb.reshape(N).astype(jnp.int32)
    lse = _compute_lse(table).reshape(V)
    tab8 = jnp.pad(table, ((0, 0), (0, NST * SW - D))).reshape(V * NST, SW)
    logitsT, parts = _make_sc_gather(V, D, N)(
        tab8, table.reshape(V * D), xbf, ybf, lse)
    loss = jnp.sum(parts) / N
    return (logitsT.T, loss)


# speedup vs baseline: 3.5348x; 2.6235x over previous
import functools

import jax
import jax.numpy as jnp
from jax import lax
from jax.experimental import pallas as pl
from jax.experimental.pallas import tpu as pltpu
from jax.experimental.pallas import tpu_sc as plsc


def _lse_body(table_ref, out_ref):
    x = table_ref[...]
    m = jnp.max(x, axis=1, keepdims=True)
    s = jnp.sum(jnp.exp(x - m), axis=1, keepdims=True)
    out_ref[...] = m + jnp.log(s)


def _compute_lse(table):
    v = table.shape[0]
    return pl.pallas_call(
        _lse_body,
        out_shape=jax.ShapeDtypeStruct((v, 1), jnp.float32),
    )(table)


@functools.lru_cache(maxsize=None)
def _make_sc_gather(V, D, B):
    info = plsc.get_sparse_core_info()
    NC, NS, L = info.num_cores, info.num_subcores, info.num_lanes
    NW = NC * NS
    assert B % NW == 0
    bpw = B // NW
    CH = 32
    assert bpw % CH == 0 and CH % L == 0
    NIT = bpw // CH
    NTILE = (D + 127) // 128
    DP = NTILE * 128
    FULLW = 128 * (NTILE - 1)
    TW = D - FULLW
    tail_segs = list(range(0, TW - L + 1, L))
    if tail_segs[-1] != TW - L:
        tail_segs.append(TW - L)
    mesh = plsc.VectorSubcoreMesh(core_axis_name="c", subcore_axis_name="s")

    @functools.partial(
        pl.kernel,
        mesh=mesh,
        out_type=[
            jax.ShapeDtypeStruct((B, D), jnp.float32),
            jax.ShapeDtypeStruct((NW * L,), jnp.float32),
        ],
        scratch_types=[
            pltpu.VMEM((2, CH), jnp.int32),
            pltpu.VMEM((2, CH), jnp.int32),
            pltpu.VMEM((2, CH), jnp.int32),
            pltpu.VMEM((2 * CH, DP), jnp.float32),
            pltpu.VMEM((CH, TW), jnp.float32),
            pltpu.VMEM((2, CH), jnp.float32),
            pltpu.VMEM((2, CH), jnp.float32),
            pltpu.VMEM((L,), jnp.float32),
            pltpu.SemaphoreType.DMA,
            pltpu.SemaphoreType.DMA,
            pltpu.SemaphoreType.DMA,
            pltpu.SemaphoreType.DMA,
        ],
    )
    def sc_kernel(tab_hbm, tflat_hbm, xb_hbm, yb_hbm, lse_hbm,
                  out_hbm, part_hbm,
                  xidx_v, yidx_v, fidx_v, rows_v, tail_v, lsev_v, tv_v,
                  acc_v, sem0, sem1, wsem0, wsem1):
        wid = lax.axis_index("s") * NC + lax.axis_index("c")
        acc_v[...] = jnp.zeros((L,), jnp.float32)
        sems = (sem0, sem1)
        wsems = (wsem0, wsem1)

        def out_copy(slot, chunk_i):
            base = wid * bpw + chunk_i * CH
            return pltpu.make_async_copy(
                rows_v.at[pl.ds(slot * CH, CH), pl.ds(0, FULLW)],
                out_hbm.at[pl.ds(base, CH), pl.ds(0, FULLW)], wsems[slot])

        def fire(slot, chunk_i):
            base = wid * bpw + chunk_i * CH

            @pl.when(chunk_i >= 2)
            def _():
                out_copy(slot, chunk_i - 2).wait()

            pltpu.sync_copy(xb_hbm.at[pl.ds(base, CH)], xidx_v.at[slot])
            pltpu.sync_copy(yb_hbm.at[pl.ds(base, CH)], yidx_v.at[slot])
            for kk in range(CH // L):
                sl = pl.ds(kk * L, L)
                fidx_v[slot, sl] = xidx_v[slot, sl] * D + yidx_v[slot, sl]
            pltpu.async_copy(
                tab_hbm.at[xidx_v.at[slot]],
                rows_v.at[pl.ds(slot * CH, CH)], sems[slot])
            pltpu.async_copy(
                lse_hbm.at[xidx_v.at[slot]], lsev_v.at[slot], sems[slot])
            pltpu.async_copy(
                tflat_hbm.at[fidx_v.at[slot]], tv_v.at[slot], sems[slot])

        def drain(slot):
            pltpu.make_async_copy(
                tab_hbm.at[xidx_v.at[slot]],
                rows_v.at[pl.ds(slot * CH, CH)], sems[slot]).wait()
            pltpu.make_async_copy(
                lse_hbm.at[xidx_v.at[slot]], lsev_v.at[slot],
                sems[slot]).wait()
            pltpu.make_async_copy(
                tflat_hbm.at[fidx_v.at[slot]], tv_v.at[slot],
                sems[slot]).wait()

        def consume(slot, chunk_i):
            base = wid * bpw + chunk_i * CH
            rb = slot * CH
            out_copy(slot, chunk_i).start()
            for t in range(CH):
                for c in tail_segs:
                    tail_v[t, pl.ds(c, L)] = rows_v[rb + t, pl.ds(FULLW + c, L)]
            pltpu.sync_copy(
                tail_v, out_hbm.at[pl.ds(base, CH), pl.ds(FULLW, TW)])
            for kk in range(CH // L):
                sl = pl.ds(kk * L, L)
                acc_v[...] = acc_v[...] + (lsev_v[slot, sl] - tv_v[slot, sl])

        fire(0, 0)

        def body(i, carry):
            @pl.when(i % 2 == 0)
            def _():
                @pl.when(i + 1 < NIT)
                def _():
                    fire(1, i + 1)
                drain(0)
                consume(0, i)

            @pl.when(i % 2 == 1)
            def _():
                @pl.when(i + 1 < NIT)
                def _():
                    fire(0, i + 1)
                drain(1)
                consume(1, i)

            return carry

        lax.fori_loop(0, NIT, body, 0)
        out_copy((NIT - 2) % 2, NIT - 2).wait()
        out_copy((NIT - 1) % 2, NIT - 1).wait()
        pltpu.sync_copy(acc_v, part_hbm.at[pl.ds(wid * L, L)])

    return sc_kernel


def kernel(xb, yb, table):
    Bb, Tt = xb.shape
    V, D = table.shape
    N = Bb * Tt
    NTILE = (D + 127) // 128
    xbf = xb.reshape(N).astype(jnp.int32)
    ybf = yb.reshape(N).astype(jnp.int32)
    lse = _compute_lse(table).reshape(V)
    tab = jnp.pad(table, ((0, 0), (0, NTILE * 128 - D)))
    logits2, parts = _make_sc_gather(V, D, N)(
        tab, table.reshape(V * D), xbf, ybf, lse)
    loss = jnp.sum(parts) / N
    return (logits2, loss)
